# baseline (speedup 1.0000x reference)
import dataclasses

import jax
import jax.numpy as jnp
from jax import lax
from jax.experimental import pallas as pl
from jax.experimental.pallas import tpu as pltpu
from jax.experimental.pallas import tpu_sc as plsc

N_CLASSES = 1000
BATCH = 16384
NUM_WORKERS = 32
ROWS_PER_WORKER = BATCH // NUM_WORKERS
BLOCK_ROWS = 64
NUM_BLOCKS = ROWS_PER_WORKER // BLOCK_ROWS
BLOCK_ELEMS = BLOCK_ROWS * N_CLASSES
LANES = 16


def _onehot_flat(x):
    mesh = plsc.VectorSubcoreMesh(core_axis_name="c", subcore_axis_name="s")
    cp = pltpu.CompilerParams()
    if "needs_layout_passes" in pltpu.CompilerParams.__dataclass_fields__:
        cp = dataclasses.replace(cp, needs_layout_passes=False)

    @pl.kernel(
        compiler_params=cp,
        out_type=jax.ShapeDtypeStruct((BATCH * N_CLASSES,), jnp.float32),
        mesh=mesh,
        scratch_types=[
            pltpu.VMEM((ROWS_PER_WORKER,), jnp.int32),
            pltpu.VMEM((BLOCK_ELEMS,), jnp.float32),
            pltpu.VMEM((BLOCK_ELEMS,), jnp.float32),
            pltpu.SemaphoreType.DMA,
            pltpu.SemaphoreType.DMA,
        ],
    )
    def body(x_hbm, out_hbm, idx_v, buf_a, buf_b, sem_a, sem_b):
        wid = lax.axis_index("s") * 2 + lax.axis_index("c")
        row0 = wid * ROWS_PER_WORKER

        pltpu.sync_copy(x_hbm.at[pl.ds(row0, ROWS_PER_WORKER)], idx_v)

        zeros16 = jnp.zeros((LANES,), jnp.float32)
        ones16 = jnp.ones((LANES,), jnp.float32)
        lane_iota = lax.iota(jnp.int32, LANES)

        @pl.loop(0, BLOCK_ELEMS, step=LANES)
        def _(i):
            buf_a[pl.ds(i, LANES)] = zeros16
            buf_b[pl.ds(i, LANES)] = zeros16

        out_base = row0 * N_CLASSES

        def poke(buf, b, val16):
            for j in range(BLOCK_ROWS // LANES):
                cols = idx_v[pl.ds(b * BLOCK_ROWS + j * LANES, LANES)]
                flat = (lane_iota + j * LANES) * N_CLASSES + cols
                plsc.store_scatter(buf, [flat], val16)

        def start(buf, b, sem):
            dst = out_hbm.at[pl.ds(out_base + b * BLOCK_ELEMS, BLOCK_ELEMS)]
            pltpu.async_copy(buf, dst, sem)

        def wait(buf, sem):
            dst = out_hbm.at[pl.ds(out_base, BLOCK_ELEMS)]
            pltpu.make_async_copy(buf, dst, sem).wait()

        poke(buf_a, 0, ones16)
        start(buf_a, 0, sem_a)
        poke(buf_b, 1, ones16)
        start(buf_b, 1, sem_b)

        @pl.loop(0, NUM_BLOCKS // 2 - 1)
        def _(i):
            b = 2 * i
            wait(buf_a, sem_a)
            poke(buf_a, b, zeros16)
            poke(buf_a, b + 2, ones16)
            start(buf_a, b + 2, sem_a)
            wait(buf_b, sem_b)
            poke(buf_b, b + 1, zeros16)
            poke(buf_b, b + 3, ones16)
            start(buf_b, b + 3, sem_b)

        wait(buf_a, sem_a)
        wait(buf_b, sem_b)

    return body(x)


def kernel(x):
    out = _onehot_flat(x.astype(jnp.int32))
    return out.reshape(BATCH, N_CLASSES)


# speedup vs baseline: 1.7787x; 1.7787x over previous
import dataclasses

import jax
import jax.numpy as jnp
from jax import lax
from jax.experimental import pallas as pl
from jax.experimental.pallas import tpu as pltpu
from jax.experimental.pallas import tpu_sc as plsc

N_CLASSES = 1000
BATCH = 16384
NUM_WORKERS = 32
ROWS_PER_WORKER = BATCH // NUM_WORKERS
BLOCK_ROWS = 32
NUM_BLOCKS = ROWS_PER_WORKER // BLOCK_ROWS
LANES = 16


def _onehot(x):
    mesh = plsc.VectorSubcoreMesh(core_axis_name="c", subcore_axis_name="s")
    cp = pltpu.CompilerParams()
    if "needs_layout_passes" in pltpu.CompilerParams.__dataclass_fields__:
        cp = dataclasses.replace(cp, needs_layout_passes=False)

    @pl.kernel(
        compiler_params=cp,
        out_type=jax.ShapeDtypeStruct((BATCH, N_CLASSES), jnp.float32),
        mesh=mesh,
        scratch_types=[
            pltpu.VMEM((ROWS_PER_WORKER,), jnp.int32),
            pltpu.VMEM((BLOCK_ROWS, N_CLASSES), jnp.float32),
            pltpu.VMEM((BLOCK_ROWS, N_CLASSES), jnp.float32),
            pltpu.SemaphoreType.DMA,
            pltpu.SemaphoreType.DMA,
        ],
    )
    def body(x_hbm, out_hbm, idx_v, buf_a, buf_b, sem_a, sem_b):
        wid = lax.axis_index("s") * 2 + lax.axis_index("c")
        row0 = wid * ROWS_PER_WORKER

        pltpu.sync_copy(x_hbm.at[pl.ds(row0, ROWS_PER_WORKER)], idx_v)

        zeros16 = jnp.zeros((LANES,), jnp.float32)
        ones16 = jnp.ones((LANES,), jnp.float32)
        lane_iota = lax.iota(jnp.int32, LANES)

        @pl.loop(0, BLOCK_ROWS)
        def _(r):
            for c in list(range(0, N_CLASSES - LANES, LANES)) + [N_CLASSES - LANES]:
                buf_a[r, pl.ds(c, LANES)] = zeros16
                buf_b[r, pl.ds(c, LANES)] = zeros16

        def poke(buf, b, val16):
            for j in range(BLOCK_ROWS // LANES):
                cols = idx_v[pl.ds(b * BLOCK_ROWS + j * LANES, LANES)]
                rows = lane_iota + j * LANES
                plsc.store_scatter(buf, [rows, cols], val16)

        def start(buf, b, sem):
            dst = out_hbm.at[pl.ds(row0 + b * BLOCK_ROWS, BLOCK_ROWS)]
            pltpu.async_copy(buf, dst, sem)

        def wait(buf, sem):
            dst = out_hbm.at[pl.ds(row0, BLOCK_ROWS)]
            pltpu.make_async_copy(buf, dst, sem).wait()

        poke(buf_a, 0, ones16)
        start(buf_a, 0, sem_a)
        poke(buf_b, 1, ones16)
        start(buf_b, 1, sem_b)

        @pl.loop(0, NUM_BLOCKS // 2 - 1)
        def _(i):
            b = 2 * i
            wait(buf_a, sem_a)
            poke(buf_a, b, zeros16)
            poke(buf_a, b + 2, ones16)
            start(buf_a, b + 2, sem_a)
            wait(buf_b, sem_b)
            poke(buf_b, b + 1, zeros16)
            poke(buf_b, b + 3, ones16)
            start(buf_b, b + 3, sem_b)

        wait(buf_a, sem_a)
        wait(buf_b, sem_b)

    return body(x)


def kernel(x):
    return _onehot(x.astype(jnp.int32))
